# baseline (speedup 1.0000x reference)
import functools

import numpy as np
import jax
import jax.numpy as jnp
from jax import lax
from jax.experimental import pallas as pl
from jax.experimental.pallas import tpu as pltpu
from jax.experimental.pallas import tpu_sc as plsc

_D = 1024
_B = 4
_S = 2048
_NC, _NS, _L = 2, 16, 16
_NW = _NC * _NS
_ROWS = _B * _S
_RPW = _ROWS // _NW
_CH = 40
_NB = 3
_SIZES = [40, 40, 40, 40, 40, 40, 16]
_OFFS = [sum(_SIZES[:i]) for i in range(len(_SIZES))]
_NCHUNK = len(_SIZES)
_GPR = _D // _L


def _pos_fix_np():
    pos = np.arange(_S, dtype=np.float32)
    fix = np.zeros((_S, _L), dtype=np.float32)
    fix[:, 0] = np.sin(pos)
    fix[:, 1] = np.cos(pos) - np.float32(1.0)
    return fix.reshape(-1)


_POS_FIX = _pos_fix_np()

_mesh = plsc.VectorSubcoreMesh(core_axis_name="c", subcore_axis_name="s")


@functools.partial(
    pl.kernel,
    out_type=jax.ShapeDtypeStruct((_B, _S, _D), jnp.float32),
    mesh=_mesh,
    scratch_types=[
        pltpu.VMEM((_RPW,), jnp.int32),
        pltpu.VMEM((_RPW * _L,), jnp.float32),
        [pltpu.VMEM((_CH, _D), jnp.float32)] * _NB,
        [pltpu.SemaphoreType.DMA] * _NB,
        [pltpu.SemaphoreType.DMA] * _NB,
        pltpu.SemaphoreType.DMA,
    ],
)
def _emb_kernel(x_hbm, pos_hbm, table_hbm, out_hbm,
                idx_v, pos_v, bufs, g_sems, s_sems, p_sem):
    wid = lax.axis_index("s") * _NC + lax.axis_index("c")
    bi = wid // (_S // _RPW)
    pbase = lax.rem(wid, _S // _RPW) * _RPW

    pltpu.sync_copy(x_hbm.at[bi, pl.ds(pbase, _RPW)], idx_v)
    pos_cp = pltpu.async_copy(
        pos_hbm.at[pl.ds(pbase * _L, _RPW * _L)], pos_v, p_sem)

    pattern = lax.rem(lax.iota(jnp.int32, 16), 2).astype(jnp.float32)

    def start_gather(c):
        b = c % _NB
        return pltpu.async_copy(
            table_hbm.at[idx_v.at[pl.ds(_OFFS[c], _SIZES[c])]],
            bufs[b].at[pl.ds(0, _SIZES[c])], g_sems[b])

    gathers = [None] * _NCHUNK
    stores = [None] * _NCHUNK
    for c in range(_NB - 1):
        gathers[c] = start_gather(c)
    for c in range(_NCHUNK):
        b = c % _NB
        buf = bufs[b]
        k = c + _NB - 1
        if k < _NCHUNK:
            if k - _NB >= 0:
                stores[k - _NB].wait()
            gathers[k] = start_gather(k)
        gathers[c].wait()
        if c == 0:
            pos_cp.wait()

        def row_pair_body(i, _):
            for u in range(2):
                r = i * 2 + u
                fix = pos_v[pl.ds((_OFFS[c] + r) * _L, _L)] + pattern
                plsc.addupdate(buf.at[r, pl.ds(0, _L)], fix)
                for g in range(1, _GPR):
                    plsc.addupdate(buf.at[r, pl.ds(g * _L, _L)], pattern)
            return _

        lax.fori_loop(0, _SIZES[c] // 2, row_pair_body, None)

        stores[c] = pltpu.async_copy(
            bufs[b].at[pl.ds(0, _SIZES[c])],
            out_hbm.at[bi, pl.ds(pbase + _OFFS[c], _SIZES[c])], s_sems[b])
    for c in range(_NCHUNK - _NB, _NCHUNK):
        stores[c].wait()


def kernel(x, table):
    pos_fix = jnp.asarray(_POS_FIX)
    return _emb_kernel(x, pos_fix, table)


# speedup vs baseline: 1.0291x; 1.0291x over previous
import functools

import numpy as np
import jax
import jax.numpy as jnp
from jax import lax
from jax.experimental import pallas as pl
from jax.experimental.pallas import tpu as pltpu
from jax.experimental.pallas import tpu_sc as plsc

_D = 1024
_B = 4
_S = 2048
_NC, _NS, _L = 2, 16, 16
_NW = _NC * _NS
_ROWS = _B * _S
_RPW = _ROWS // _NW
_CH = 40
_NB = 3
_SIZES = [40, 40, 40, 40, 40, 40, 16]
_OFFS = [sum(_SIZES[:i]) for i in range(len(_SIZES))]
_NCHUNK = len(_SIZES)
_GPR = _D // _L


def _pos_fix_np():
    pos = np.arange(_S, dtype=np.float32)
    fix = np.zeros((_S, _L), dtype=np.float32)
    fix[:, 0] = np.sin(pos)
    fix[:, 1] = np.cos(pos) - np.float32(1.0)
    return fix.reshape(-1)


_POS_FIX = _pos_fix_np()

_mesh = plsc.VectorSubcoreMesh(core_axis_name="c", subcore_axis_name="s")


@functools.partial(
    pl.kernel,
    out_type=jax.ShapeDtypeStruct((_B, _S, _D), jnp.float32),
    mesh=_mesh,
    scratch_types=[
        pltpu.VMEM((_RPW,), jnp.int32),
        pltpu.VMEM((_RPW * _L,), jnp.float32),
        [pltpu.VMEM((_CH, _D), jnp.float32)] * _NB,
        [pltpu.SemaphoreType.DMA] * _NB,
        [pltpu.SemaphoreType.DMA] * _NB,
        pltpu.SemaphoreType.DMA,
    ],
)
def _emb_kernel(x_hbm, pos_hbm, table_hbm, out_hbm,
                idx_v, pos_v, bufs, g_sems, s_sems, p_sem):
    wid = lax.axis_index("s") * _NC + lax.axis_index("c")
    bi = wid // (_S // _RPW)
    pbase = lax.rem(wid, _S // _RPW) * _RPW

    pltpu.sync_copy(x_hbm.at[bi, pl.ds(pbase, _RPW)], idx_v)
    pos_cp = pltpu.async_copy(
        pos_hbm.at[pl.ds(pbase * _L, _RPW * _L)], pos_v, p_sem)

    pattern = lax.rem(lax.iota(jnp.int32, 16), 2).astype(jnp.float32)

    def start_gather(c):
        b = c % _NB
        return pltpu.async_copy(
            table_hbm.at[idx_v.at[pl.ds(_OFFS[c], _SIZES[c])]],
            bufs[b].at[pl.ds(0, _SIZES[c])], g_sems[b])

    gathers = [None] * _NCHUNK
    stores = [None] * _NCHUNK
    for c in range(_NB - 1):
        gathers[c] = start_gather(c)
    for c in range(_NCHUNK):
        b = c % _NB
        buf = bufs[b]
        k = c + _NB - 1
        if k < _NCHUNK:
            if k - _NB >= 0:
                stores[k - _NB].wait()
            gathers[k] = start_gather(k)
        gathers[c].wait()
        if c == 0:
            pos_cp.wait()

        def row_body(r, _):
            fix = pos_v[pl.ds((_OFFS[c] + r) * _L, _L)] + pattern
            plsc.addupdate(buf.at[r, pl.ds(0, _L)], fix)
            for g in range(1, _GPR):
                plsc.addupdate(buf.at[r, pl.ds(g * _L, _L)], pattern)
            return _

        lax.fori_loop(0, _SIZES[c], row_body, None)

        stores[c] = pltpu.async_copy(
            bufs[b].at[pl.ds(0, _SIZES[c])],
            out_hbm.at[bi, pl.ds(pbase + _OFFS[c], _SIZES[c])], s_sems[b])
    for c in range(_NCHUNK - _NB, _NCHUNK):
        stores[c].wait()


def kernel(x, table):
    pos_fix = jnp.asarray(_POS_FIX)
    return _emb_kernel(x, pos_fix, table)


# speedup vs baseline: 1.0479x; 1.0183x over previous
import functools

import numpy as np
import jax
import jax.numpy as jnp
from jax import lax
from jax.experimental import pallas as pl
from jax.experimental.pallas import tpu as pltpu
from jax.experimental.pallas import tpu_sc as plsc

_D = 1024
_B = 4
_S = 2048
_NC, _NS, _L = 2, 16, 16
_NW = _NC * _NS
_ROWS = _B * _S
_RPW = _ROWS // _NW
_CH = 40
_NB = 3
_SIZES = [16, 40, 40, 40, 40, 40, 40]
_OFFS = [sum(_SIZES[:i]) for i in range(len(_SIZES))]
_NCHUNK = len(_SIZES)
_GPR = _D // _L


def _pos_fix_np():
    pos = np.arange(_S, dtype=np.float32)
    fix = np.zeros((_S, _L), dtype=np.float32)
    fix[:, 0] = np.sin(pos)
    fix[:, 1] = np.cos(pos) - np.float32(1.0)
    return fix.reshape(-1)


_POS_FIX = _pos_fix_np()

_mesh = plsc.VectorSubcoreMesh(core_axis_name="c", subcore_axis_name="s")


@functools.partial(
    pl.kernel,
    out_type=jax.ShapeDtypeStruct((_B, _S, _D), jnp.float32),
    mesh=_mesh,
    scratch_types=[
        pltpu.VMEM((_RPW,), jnp.int32),
        pltpu.VMEM((_RPW * _L,), jnp.float32),
        [pltpu.VMEM((_CH, _D), jnp.float32)] * _NB,
        [pltpu.SemaphoreType.DMA] * _NB,
        [pltpu.SemaphoreType.DMA] * _NB,
        pltpu.SemaphoreType.DMA,
    ],
)
def _emb_kernel(x_hbm, pos_hbm, table_hbm, out_hbm,
                idx_v, pos_v, bufs, g_sems, s_sems, p_sem):
    wid = lax.axis_index("s") * _NC + lax.axis_index("c")
    bi = wid // (_S // _RPW)
    pbase = lax.rem(wid, _S // _RPW) * _RPW

    pltpu.sync_copy(x_hbm.at[bi, pl.ds(pbase, _RPW)], idx_v)
    pos_cp = pltpu.async_copy(
        pos_hbm.at[pl.ds(pbase * _L, _RPW * _L)], pos_v, p_sem)

    pattern = lax.rem(lax.iota(jnp.int32, 16), 2).astype(jnp.float32)

    def start_gather(c):
        b = c % _NB
        return pltpu.async_copy(
            table_hbm.at[idx_v.at[pl.ds(_OFFS[c], _SIZES[c])]],
            bufs[b].at[pl.ds(0, _SIZES[c])], g_sems[b])

    gathers = [None] * _NCHUNK
    stores = [None] * _NCHUNK
    for c in range(_NB - 1):
        gathers[c] = start_gather(c)
    for c in range(_NCHUNK):
        b = c % _NB
        buf = bufs[b]
        k = c + _NB - 1
        if k < _NCHUNK:
            if k - _NB >= 0:
                stores[k - _NB].wait()
            gathers[k] = start_gather(k)
        gathers[c].wait()
        if c == 0:
            pos_cp.wait()

        def row_body(r, _):
            fix = pos_v[pl.ds((_OFFS[c] + r) * _L, _L)] + pattern
            plsc.addupdate(buf.at[r, pl.ds(0, _L)], fix)
            for g in range(1, _GPR):
                plsc.addupdate(buf.at[r, pl.ds(g * _L, _L)], pattern)
            return _

        lax.fori_loop(0, _SIZES[c], row_body, None)

        stores[c] = pltpu.async_copy(
            bufs[b].at[pl.ds(0, _SIZES[c])],
            out_hbm.at[bi, pl.ds(pbase + _OFFS[c], _SIZES[c])], s_sems[b])
    for c in range(_NCHUNK - _NB, _NCHUNK):
        stores[c].wait()


def kernel(x, table):
    pos_fix = jnp.asarray(_POS_FIX)
    return _emb_kernel(x, pos_fix, table)
